# baseline (speedup 1.0000x reference)
import jax
import jax.numpy as jnp
from jax import lax
from jax.experimental import pallas as pl
from jax.experimental.pallas import tpu as pltpu
from jax.experimental.pallas import tpu_sc as plsc

N = 256
D = 128
HEADS = 4
DH = D // HEADS
BT = 64
ROW = D * BT
NC, NS, L = 2, 16, 16
NW = NC * NS
NPW = N // NW
CH = 2
MAXDEG = 40
NBLK = 32


def _tc_proj_kernel(h_ref, wl_ref, wv_ref, xq_ref, xqv_ref):
    for t in range(NBLK):
        hn = h_ref[:, t, :]
        xq = lax.dot_general(wl_ref[...], hn, (((1,), (1,)), ((), ())),
                             preferred_element_type=jnp.float32)
        xv = lax.dot_general(wv_ref[...], hn, (((1,), (1,)), ((), ())),
                             preferred_element_type=jnp.float32)
        xq_ref[t] = xq
        xqv_ref[t, 0] = xq
        xqv_ref[t, 1] = xv


def _tc_out_kernel(y_ref, wo_ref, out_ref):
    for t in range(NBLK):
        out_ref[:, t, :] = lax.dot_general(
            y_ref[t], wo_ref[...], (((0,), (1,)), ((), ())),
            preferred_element_type=jnp.float32)


def _sc_edge_body(xq_hbm, xqv_hbm, a0_hbm, ab_hbm, src_hbm, dst_hbm, y_hbm,
                  src_v, dst_v, ab_v, a0r, qrow, kbuf, ybuf, denb, ixst,
                  sm, dsem):
    wid = lax.axis_index("s") * NC + lax.axis_index("c")
    base_node = wid * NPW
    epad = src_v.shape[0]
    lane0 = lax.iota(jnp.int32, 16) == 0
    zi16 = jnp.zeros((16,), jnp.int32)
    zf16v = jnp.zeros((16,), jnp.float32)

    def _lane0i(v):
        return jnp.max(jnp.where(lane0, v, zi16))

    def _lane0f(v):
        return jnp.max(jnp.where(lane0, v, zf16v))

    pltpu.sync_copy(src_hbm, src_v)
    pltpu.sync_copy(dst_hbm, dst_v)
    pltpu.sync_copy(ab_hbm, ab_v)
    pltpu.sync_copy(a0_hbm.at[pl.ds(base_node, NPW)], a0r)

    one16 = jnp.ones((16,), jnp.float32)
    zero16 = jnp.zeros((16,), jnp.float32)

    def cnt_body(i, carry):
        v = src_v[pl.ds(i * 16, 16)]
        c0 = carry[0] + jnp.where(v < base_node, one16, zero16)
        cts = tuple(carry[1 + t] + jnp.where(v == base_node + t, one16, zero16)
                    for t in range(NPW))
        return (c0,) + cts
    zeros_i = tuple(jnp.zeros((16,), jnp.float32) for _ in range(NPW + 1))
    carry = lax.fori_loop(0, epad // 16, cnt_body, zeros_i)
    run = jnp.sum(carry[0]).astype(jnp.int32)
    for t in range(NPW):
        sm[t] = run
        run = run + jnp.sum(carry[1 + t]).astype(jnp.int32)
    sm[NPW] = run

    def node_body(nt, _):
        est = sm[nt]
        deg = sm[nt + 1] - est
        node = base_node + nt
        pltpu.sync_copy(xq_hbm.at[pl.ds(node, 1)], qrow)

        @plsc.parallel_loop(0, ROW // 16, 1, unroll=8)
        def z_body(i):
            ybuf[0, pl.ds(i * 16, 16)] = jnp.zeros((16,), jnp.float32)

        nch = lax.shift_right_logical(deg + (CH - 1), 1)

        def issue(c, slot):
            ixst[slot] = dst_v[pl.ds(est + c * CH, 16)]
            pltpu.async_copy(
                xqv_hbm.at[ixst.at[slot, pl.ds(0, CH)]],
                kbuf.at[slot], dsem)

        def wait(slot):
            pltpu.make_async_copy(
                xqv_hbm.at[ixst.at[slot, pl.ds(0, CH)]],
                kbuf.at[slot], dsem).wait()

        def process(c, slot):
            for ce in range(CH):
                eidx = c * CH + ce

                @pl.when(eidx < deg)
                def _():
                    def d_body(d, acc):
                        new = list(acc)
                        dbase = d * (HEADS * BT)
                        for h in range(HEADS):
                            cvec = ab_v[d * HEADS + h]
                            for b in range(BT // 16):
                                off = dbase + h * BT + 16 * b
                                q = qrow[0, pl.ds(off, 16)]
                                k = kbuf[slot, ce, pl.ds(off, 16)]
                                tt = q + k
                                u = jnp.maximum(tt, 0.2 * tt)
                                new[h * 4 + b] = new[h * 4 + b] + cvec * u
                        return tuple(new)
                    zf = tuple(jnp.zeros((16,), jnp.float32)
                               for _ in range(16))
                    acc = plsc.parallel_loop(0, DH, 1, unroll=4,
                                             carry=zf)(d_body)
                    dnode = _lane0i(dst_v[pl.ds(est + eidx, 16)])
                    w = _lane0f(a0r[nt, pl.ds(dnode, 16)]) + 1e-8
                    p = [w * jnp.exp(jnp.minimum(acc[hb], 80.0))
                         for hb in range(16)]
                    for hb in range(16):
                        plsc.addupdate(denb.at[0, pl.ds(hb * 16, 16)], p[hb])

                    @plsc.parallel_loop(0, DH, 1, unroll=4)
                    def yd_body(d):
                        dbase = d * (HEADS * BT)
                        for h in range(HEADS):
                            for b in range(BT // 16):
                                off = dbase + h * BT + 16 * b
                                v = kbuf[slot, ce, pl.ds(ROW + off, 16)]
                                plsc.addupdate(ybuf.at[0, pl.ds(off, 16)],
                                               p[h * 4 + b] * v)

        for hb in range(16):
            denb[0, pl.ds(hb * 16, 16)] = jnp.zeros((16,), jnp.float32)

        @pl.when(nch > 0)
        def _():
            issue(jnp.int32(0), 0)

        def pair_body(pidx, _c):
            c0 = pidx * 2
            wait(0)

            @pl.when(c0 + 1 < nch)
            def _():
                issue(c0 + 1, 1)
            process(c0, 0)

            @pl.when(c0 + 1 < nch)
            def _():
                wait(1)

                @pl.when(c0 + 2 < nch)
                def _():
                    issue(c0 + 2, 0)
                process(c0 + 1, 1)
            return 0
        lax.fori_loop(0, lax.shift_right_logical(nch + 1, 1), pair_body, 0)

        den = [denb[0, pl.ds(hb * 16, 16)] for hb in range(16)]
        rden = [1.0 / jnp.maximum(den[hb], 1e-30) for hb in range(16)]

        @plsc.parallel_loop(0, DH, 1, unroll=4)
        def norm_body(d):
            dbase = d * (HEADS * BT)
            for h in range(HEADS):
                for b in range(BT // 16):
                    off = dbase + h * BT + 16 * b
                    ybuf[0, pl.ds(off, 16)] = (ybuf[0, pl.ds(off, 16)]
                                               * rden[h * 4 + b])
        pltpu.sync_copy(ybuf, y_hbm.at[pl.ds(node, 1)])
        return 0

    lax.fori_loop(0, NPW, node_body, 0)


def kernel(H, W_lin, W_val, a, W_out, A0, src, dst):
    W_lin = W_lin.reshape(HEADS, DH, D).transpose(1, 0, 2).reshape(D, D)
    W_val = W_val.reshape(HEADS, DH, D).transpose(1, 0, 2).reshape(D, D)
    W_out = W_out.reshape(D, HEADS, DH).transpose(0, 2, 1).reshape(D, D)
    a = a.T
    e = src.shape[0]
    epad = ((e + 31) // 16) * 16
    srcp = jnp.full((epad,), 4 * N, jnp.int32).at[:e].set(src)
    dstp = jnp.zeros((epad,), jnp.int32).at[:e].set(dst)

    xqT, xqvT = pl.pallas_call(
        _tc_proj_kernel,
        grid=(N // NBLK,),
        in_specs=[pl.BlockSpec((BT, NBLK, D), lambda i: (0, i, 0)),
                  pl.BlockSpec((D, D), lambda i: (0, 0)),
                  pl.BlockSpec((D, D), lambda i: (0, 0))],
        out_specs=[pl.BlockSpec((NBLK, D, BT), lambda i: (i, 0, 0)),
                   pl.BlockSpec((NBLK, 2, D, BT), lambda i: (i, 0, 0, 0))],
        out_shape=[jax.ShapeDtypeStruct((N, D, BT), jnp.float32),
                   jax.ShapeDtypeStruct((N, 2, D, BT), jnp.float32)],
    )(H, W_lin, W_val)

    y_fn = pl.kernel(
        _sc_edge_body,
        out_type=jax.ShapeDtypeStruct((N, ROW), jnp.float32),
        mesh=plsc.VectorSubcoreMesh(core_axis_name="c", subcore_axis_name="s",
                                    num_cores=NC, num_subcores=NS),
        compiler_params=pltpu.CompilerParams(needs_layout_passes=False),
        scratch_types=[
            pltpu.VMEM((epad,), jnp.int32),
            pltpu.VMEM((epad,), jnp.int32),
            pltpu.VMEM((D, 16), jnp.float32),
            pltpu.VMEM((NPW, N + 16), jnp.float32),
            pltpu.VMEM((1, ROW), jnp.float32),
            pltpu.VMEM((2, CH, 2 * ROW), jnp.float32),
            pltpu.VMEM((1, ROW), jnp.float32),
            pltpu.VMEM((1, 256), jnp.float32),
            pltpu.VMEM((2, 16), jnp.int32),
            pltpu.SMEM((16,), jnp.int32),
            pltpu.SemaphoreType.DMA,
        ],
    )
    abct = jnp.broadcast_to(a.reshape(D)[:, None], (D, 16))
    a0p = jnp.concatenate([A0, jnp.zeros((N, 16), jnp.float32)], axis=1)
    y = y_fn(xqT.reshape(N, ROW), xqvT.reshape(N, 2 * ROW), a0p, abct,
             srcp, dstp)

    out = pl.pallas_call(
        _tc_out_kernel,
        grid=(N // NBLK,),
        in_specs=[pl.BlockSpec((NBLK, D, BT), lambda i: (i, 0, 0)),
                  pl.BlockSpec((D, D), lambda i: (0, 0))],
        out_specs=pl.BlockSpec((BT, NBLK, D), lambda i: (0, i, 0)),
        out_shape=jax.ShapeDtypeStruct((BT, N, D), jnp.float32),
    )(y.reshape(N, D, BT), W_out)
    return out


# speedup vs baseline: 1.7306x; 1.7306x over previous
import jax
import jax.numpy as jnp
from jax import lax
from jax.experimental import pallas as pl
from jax.experimental.pallas import tpu as pltpu
from jax.experimental.pallas import tpu_sc as plsc

N = 256
D = 128
HEADS = 4
DH = D // HEADS
BT = 64
ROW = D * BT
NC, NS, L = 2, 16, 16
NW = NC * NS
NPW = N // NW
CH = 2
MAXDEG = 40
NBLK = 32


def _tc_proj_kernel(h_ref, wl_ref, wv_ref, xq_ref, xqv_ref):
    for t in range(NBLK):
        hn = h_ref[:, t, :]
        xq = lax.dot_general(wl_ref[...], hn, (((1,), (1,)), ((), ())),
                             preferred_element_type=jnp.float32)
        xv = lax.dot_general(wv_ref[...], hn, (((1,), (1,)), ((), ())),
                             preferred_element_type=jnp.float32)
        xq_ref[t] = xq
        xqv_ref[t, 0] = xq
        xqv_ref[t, 1] = xv


def _tc_out_kernel(y_ref, wo_ref, out_ref):
    for t in range(NBLK):
        out_ref[:, t, :] = lax.dot_general(
            y_ref[t], wo_ref[...], (((0,), (1,)), ((), ())),
            preferred_element_type=jnp.float32)


def _sc_edge_body(xq_hbm, xqv_hbm, a0_hbm, ab_hbm, src_hbm, dst_hbm, y_hbm,
                  src_v, dst_v, ab_v, a0r, qrow, kbuf, ybuf, denb, ixst,
                  sm, dsem):
    wid = lax.axis_index("s") * NC + lax.axis_index("c")
    base_node = wid * NPW
    epad = src_v.shape[0]
    lane0 = lax.iota(jnp.int32, 16) == 0
    zi16 = jnp.zeros((16,), jnp.int32)
    zf16v = jnp.zeros((16,), jnp.float32)

    def _lane0i(v):
        return jnp.max(jnp.where(lane0, v, zi16))

    def _lane0f(v):
        return jnp.max(jnp.where(lane0, v, zf16v))

    pltpu.sync_copy(src_hbm, src_v)
    pltpu.sync_copy(dst_hbm, dst_v)
    pltpu.sync_copy(ab_hbm, ab_v)
    pltpu.sync_copy(a0_hbm.at[pl.ds(base_node, NPW)], a0r)

    one16 = jnp.ones((16,), jnp.float32)
    zero16 = jnp.zeros((16,), jnp.float32)

    def cnt_body(i, carry):
        v = src_v[pl.ds(i * 16, 16)]
        c0 = carry[0] + jnp.where(v < base_node, one16, zero16)
        cts = tuple(carry[1 + t] + jnp.where(v == base_node + t, one16, zero16)
                    for t in range(NPW))
        return (c0,) + cts
    zeros_i = tuple(jnp.zeros((16,), jnp.float32) for _ in range(NPW + 1))
    carry = lax.fori_loop(0, epad // 16, cnt_body, zeros_i)
    run = jnp.sum(carry[0]).astype(jnp.int32)
    for t in range(NPW):
        sm[t] = run
        run = run + jnp.sum(carry[1 + t]).astype(jnp.int32)
    sm[NPW] = run

    def node_body(nt, _):
        est = sm[nt]
        deg = sm[nt + 1] - est
        node = base_node + nt
        pltpu.sync_copy(xq_hbm.at[pl.ds(node, 1)], qrow)

        @plsc.parallel_loop(0, ROW // 16, 1, unroll=8)
        def z_body(i):
            ybuf[0, pl.ds(i * 16, 16)] = jnp.zeros((16,), jnp.float32)

        nch = lax.shift_right_logical(deg + (CH - 1), 1)

        def issue(c, slot):
            ixst[slot] = dst_v[pl.ds(est + c * CH, 16)]
            pltpu.async_copy(
                xqv_hbm.at[ixst.at[slot, pl.ds(0, CH)]],
                kbuf.at[slot], dsem)

        def wait(slot):
            pltpu.make_async_copy(
                xqv_hbm.at[ixst.at[slot, pl.ds(0, CH)]],
                kbuf.at[slot], dsem).wait()

        def process(c, slot):
            for ce in range(CH):
                eidx = c * CH + ce

                @pl.when(eidx < deg)
                def _():
                    acc = []
                    zf4 = tuple(jnp.zeros((16,), jnp.float32)
                                for _ in range(4))
                    for h in range(HEADS):
                        def dh_body(d, a4, _h=h):
                            dbase = d * (HEADS * BT) + _h * BT
                            cvec = ab_v[d * HEADS + _h]
                            new = []
                            for b in range(BT // 16):
                                off = dbase + 16 * b
                                q = qrow[0, pl.ds(off, 16)]
                                k = kbuf[slot, ce, pl.ds(off, 16)]
                                tt = q + k
                                u = jnp.maximum(tt, 0.2 * tt)
                                new.append(a4[b] + cvec * u)
                            return tuple(new)
                        a4 = plsc.parallel_loop(0, DH, 1, unroll=2,
                                                carry=zf4)(dh_body)
                        acc.extend(a4)
                    dnode = _lane0i(dst_v[pl.ds(est + eidx, 16)])
                    w = _lane0f(a0r[nt, pl.ds(dnode, 16)]) + 1e-8
                    p = [w * jnp.exp(jnp.minimum(acc[hb], 80.0))
                         for hb in range(16)]
                    for hb in range(16):
                        plsc.addupdate(denb.at[0, pl.ds(hb * 16, 16)], p[hb])

                    @plsc.parallel_loop(0, DH, 1, unroll=4)
                    def yd_body(d):
                        dbase = d * (HEADS * BT)
                        for h in range(HEADS):
                            for b in range(BT // 16):
                                off = dbase + h * BT + 16 * b
                                v = kbuf[slot, ce, pl.ds(ROW + off, 16)]
                                plsc.addupdate(ybuf.at[0, pl.ds(off, 16)],
                                               p[h * 4 + b] * v)

        for hb in range(16):
            denb[0, pl.ds(hb * 16, 16)] = jnp.zeros((16,), jnp.float32)

        @pl.when(nch > 0)
        def _():
            issue(jnp.int32(0), 0)

        def pair_body(pidx, _c):
            c0 = pidx * 2
            wait(0)

            @pl.when(c0 + 1 < nch)
            def _():
                issue(c0 + 1, 1)
            process(c0, 0)

            @pl.when(c0 + 1 < nch)
            def _():
                wait(1)

                @pl.when(c0 + 2 < nch)
                def _():
                    issue(c0 + 2, 0)
                process(c0 + 1, 1)
            return 0
        lax.fori_loop(0, lax.shift_right_logical(nch + 1, 1), pair_body, 0)

        den = [denb[0, pl.ds(hb * 16, 16)] for hb in range(16)]
        rden = [1.0 / jnp.maximum(den[hb], 1e-30) for hb in range(16)]

        @plsc.parallel_loop(0, DH, 1, unroll=4)
        def norm_body(d):
            dbase = d * (HEADS * BT)
            for h in range(HEADS):
                for b in range(BT // 16):
                    off = dbase + h * BT + 16 * b
                    ybuf[0, pl.ds(off, 16)] = (ybuf[0, pl.ds(off, 16)]
                                               * rden[h * 4 + b])
        pltpu.sync_copy(ybuf, y_hbm.at[pl.ds(node, 1)])
        return 0

    lax.fori_loop(0, NPW, node_body, 0)


def kernel(H, W_lin, W_val, a, W_out, A0, src, dst):
    W_lin = W_lin.reshape(HEADS, DH, D).transpose(1, 0, 2).reshape(D, D)
    W_val = W_val.reshape(HEADS, DH, D).transpose(1, 0, 2).reshape(D, D)
    W_out = W_out.reshape(D, HEADS, DH).transpose(0, 2, 1).reshape(D, D)
    a = a.T
    e = src.shape[0]
    epad = ((e + 31) // 16) * 16
    srcp = jnp.full((epad,), 4 * N, jnp.int32).at[:e].set(src)
    dstp = jnp.zeros((epad,), jnp.int32).at[:e].set(dst)

    xqT, xqvT = pl.pallas_call(
        _tc_proj_kernel,
        grid=(N // NBLK,),
        in_specs=[pl.BlockSpec((BT, NBLK, D), lambda i: (0, i, 0)),
                  pl.BlockSpec((D, D), lambda i: (0, 0)),
                  pl.BlockSpec((D, D), lambda i: (0, 0))],
        out_specs=[pl.BlockSpec((NBLK, D, BT), lambda i: (i, 0, 0)),
                   pl.BlockSpec((NBLK, 2, D, BT), lambda i: (i, 0, 0, 0))],
        out_shape=[jax.ShapeDtypeStruct((N, D, BT), jnp.float32),
                   jax.ShapeDtypeStruct((N, 2, D, BT), jnp.float32)],
    )(H, W_lin, W_val)

    y_fn = pl.kernel(
        _sc_edge_body,
        out_type=jax.ShapeDtypeStruct((N, ROW), jnp.float32),
        mesh=plsc.VectorSubcoreMesh(core_axis_name="c", subcore_axis_name="s",
                                    num_cores=NC, num_subcores=NS),
        compiler_params=pltpu.CompilerParams(needs_layout_passes=False),
        scratch_types=[
            pltpu.VMEM((epad,), jnp.int32),
            pltpu.VMEM((epad,), jnp.int32),
            pltpu.VMEM((D, 16), jnp.float32),
            pltpu.VMEM((NPW, N + 16), jnp.float32),
            pltpu.VMEM((1, ROW), jnp.float32),
            pltpu.VMEM((2, CH, 2 * ROW), jnp.float32),
            pltpu.VMEM((1, ROW), jnp.float32),
            pltpu.VMEM((1, 256), jnp.float32),
            pltpu.VMEM((2, 16), jnp.int32),
            pltpu.SMEM((16,), jnp.int32),
            pltpu.SemaphoreType.DMA,
        ],
    )
    abct = jnp.broadcast_to(a.reshape(D)[:, None], (D, 16))
    a0p = jnp.concatenate([A0, jnp.zeros((N, 16), jnp.float32)], axis=1)
    y = y_fn(xqT.reshape(N, ROW), xqvT.reshape(N, 2 * ROW), a0p, abct,
             srcp, dstp)

    out = pl.pallas_call(
        _tc_out_kernel,
        grid=(N // NBLK,),
        in_specs=[pl.BlockSpec((NBLK, D, BT), lambda i: (i, 0, 0)),
                  pl.BlockSpec((D, D), lambda i: (0, 0))],
        out_specs=pl.BlockSpec((BT, NBLK, D), lambda i: (0, i, 0)),
        out_shape=jax.ShapeDtypeStruct((BT, N, D), jnp.float32),
    )(y.reshape(N, D, BT), W_out)
    return out


# speedup vs baseline: 1.9381x; 1.1199x over previous
import jax
import jax.numpy as jnp
from jax import lax
from jax.experimental import pallas as pl
from jax.experimental.pallas import tpu as pltpu
from jax.experimental.pallas import tpu_sc as plsc

N = 256
D = 128
HEADS = 4
DH = D // HEADS
BT = 64
ROW = D * BT
NC, NS, L = 2, 16, 16
NW = NC * NS
NPW = N // NW
CH = 2
MAXDEG = 40
NBLK = 32


def _tc_proj_kernel(h_ref, wl_ref, wv_ref, xq_ref, xqv_ref):
    for t in range(NBLK):
        hn = h_ref[:, t, :]
        xq = lax.dot_general(wl_ref[...], hn, (((1,), (1,)), ((), ())),
                             preferred_element_type=jnp.float32)
        xv = lax.dot_general(wv_ref[...], hn, (((1,), (1,)), ((), ())),
                             preferred_element_type=jnp.float32)
        xq_ref[t] = xq
        xqv_ref[t, 0] = xq
        xqv_ref[t, 1] = xv


def _tc_out_kernel(y_ref, wo_ref, out_ref):
    for t in range(NBLK):
        out_ref[:, t, :] = lax.dot_general(
            y_ref[t], wo_ref[...], (((0,), (1,)), ((), ())),
            preferred_element_type=jnp.float32)


def _sc_edge_body(xq_hbm, xqv_hbm, a0_hbm, ab_hbm, src_hbm, dst_hbm, y_hbm,
                  src_v, dst_v, ab_v, a0r, qrow, kbuf, ybuf, denb, ixst,
                  sm, dsem):
    wid = lax.axis_index("s") * NC + lax.axis_index("c")
    base_node = wid * NPW
    epad = src_v.shape[0]
    lane0 = lax.iota(jnp.int32, 16) == 0
    zi16 = jnp.zeros((16,), jnp.int32)
    zf16v = jnp.zeros((16,), jnp.float32)

    def _lane0i(v):
        return jnp.max(jnp.where(lane0, v, zi16))

    def _lane0f(v):
        return jnp.max(jnp.where(lane0, v, zf16v))

    pltpu.sync_copy(src_hbm, src_v)
    pltpu.sync_copy(dst_hbm, dst_v)
    pltpu.sync_copy(ab_hbm, ab_v)
    pltpu.sync_copy(a0_hbm.at[pl.ds(base_node, NPW)], a0r)

    one16 = jnp.ones((16,), jnp.float32)
    zero16 = jnp.zeros((16,), jnp.float32)

    def cnt_body(i, carry):
        v = src_v[pl.ds(i * 16, 16)]
        c0 = carry[0] + jnp.where(v < base_node, one16, zero16)
        cts = tuple(carry[1 + t] + jnp.where(v == base_node + t, one16, zero16)
                    for t in range(NPW))
        return (c0,) + cts
    zeros_i = tuple(jnp.zeros((16,), jnp.float32) for _ in range(NPW + 1))
    carry = lax.fori_loop(0, epad // 16, cnt_body, zeros_i)
    run = jnp.sum(carry[0]).astype(jnp.int32)
    for t in range(NPW):
        sm[t] = run
        run = run + jnp.sum(carry[1 + t]).astype(jnp.int32)
    sm[NPW] = run

    def node_body(nt, _):
        est = sm[nt]
        deg = sm[nt + 1] - est
        node = base_node + nt
        pltpu.sync_copy(xq_hbm.at[pl.ds(node, 1)], qrow)

        @plsc.parallel_loop(0, ROW // 16, 1, unroll=8)
        def z_body(i):
            ybuf[0, pl.ds(i * 16, 16)] = jnp.zeros((16,), jnp.float32)

        nch = lax.shift_right_logical(deg + (CH - 1), 1)

        def issue(c, slot):
            ixst[slot] = dst_v[pl.ds(est + c * CH, 16)]
            pltpu.async_copy(
                xqv_hbm.at[ixst.at[slot, pl.ds(0, CH)]],
                kbuf.at[slot], dsem)

        def wait(slot):
            pltpu.make_async_copy(
                xqv_hbm.at[ixst.at[slot, pl.ds(0, CH)]],
                kbuf.at[slot], dsem).wait()

        def process(c, slot):
            e0 = c * CH
            s0 = jnp.where(e0 < deg, 1.0, 0.0)
            s1 = jnp.where(e0 + 1 < deg, 1.0, 0.0)
            accs = []
            zf8 = tuple(jnp.zeros((16,), jnp.float32) for _ in range(8))
            for h in range(HEADS):
                def dh_body(d, a8, _h=h):
                    dbase = d * (HEADS * BT) + _h * BT
                    cvec = ab_v[d * HEADS + _h]
                    new = []
                    for b in range(BT // 16):
                        off = dbase + 16 * b
                        q = qrow[0, pl.ds(off, 16)]
                        k0 = kbuf[slot, 0, pl.ds(off, 16)]
                        k1 = kbuf[slot, 1, pl.ds(off, 16)]
                        t0 = q + k0
                        t1 = q + k1
                        u0 = jnp.maximum(t0, 0.2 * t0)
                        u1 = jnp.maximum(t1, 0.2 * t1)
                        new.append(a8[2 * b] + cvec * u0)
                        new.append(a8[2 * b + 1] + cvec * u1)
                    return tuple(new)
                a8 = plsc.parallel_loop(0, DH, 1, unroll=2,
                                        carry=zf8)(dh_body)
                accs.append(a8)
            d0 = _lane0i(dst_v[pl.ds(est + e0, 16)])
            d1 = _lane0i(dst_v[pl.ds(est + e0 + 1, 16)])
            w0 = (_lane0f(a0r[nt, pl.ds(d0, 16)]) + 1e-8) * s0
            w1 = (_lane0f(a0r[nt, pl.ds(d1, 16)]) + 1e-8) * s1
            p0 = []
            p1 = []
            for h in range(HEADS):
                for b in range(BT // 16):
                    p0.append(w0 * jnp.exp(jnp.minimum(accs[h][2 * b], 80.0)))
                    p1.append(w1 * jnp.exp(
                        jnp.minimum(accs[h][2 * b + 1], 80.0)))
            for hb in range(16):
                plsc.addupdate(denb.at[0, pl.ds(hb * 16, 16)],
                               p0[hb] + p1[hb])

            @plsc.parallel_loop(0, DH, 1, unroll=2)
            def yd_body(d):
                dbase = d * (HEADS * BT)
                for h in range(HEADS):
                    for b in range(BT // 16):
                        off = dbase + h * BT + 16 * b
                        v0 = kbuf[slot, 0, pl.ds(ROW + off, 16)]
                        v1 = kbuf[slot, 1, pl.ds(ROW + off, 16)]
                        plsc.addupdate(ybuf.at[0, pl.ds(off, 16)],
                                       p0[h * 4 + b] * v0
                                       + p1[h * 4 + b] * v1)

        for hb in range(16):
            denb[0, pl.ds(hb * 16, 16)] = jnp.zeros((16,), jnp.float32)

        @pl.when(nch > 0)
        def _():
            issue(jnp.int32(0), 0)

        def pair_body(pidx, _c):
            c0 = pidx * 2
            wait(0)

            @pl.when(c0 + 1 < nch)
            def _():
                issue(c0 + 1, 1)
            process(c0, 0)

            @pl.when(c0 + 1 < nch)
            def _():
                wait(1)

                @pl.when(c0 + 2 < nch)
                def _():
                    issue(c0 + 2, 0)
                process(c0 + 1, 1)
            return 0
        lax.fori_loop(0, lax.shift_right_logical(nch + 1, 1), pair_body, 0)

        den = [denb[0, pl.ds(hb * 16, 16)] for hb in range(16)]
        rden = [1.0 / jnp.maximum(den[hb], 1e-30) for hb in range(16)]

        @plsc.parallel_loop(0, DH, 1, unroll=4)
        def norm_body(d):
            dbase = d * (HEADS * BT)
            for h in range(HEADS):
                for b in range(BT // 16):
                    off = dbase + h * BT + 16 * b
                    ybuf[0, pl.ds(off, 16)] = (ybuf[0, pl.ds(off, 16)]
                                               * rden[h * 4 + b])
        pltpu.sync_copy(ybuf, y_hbm.at[pl.ds(node, 1)])
        return 0

    lax.fori_loop(0, NPW, node_body, 0)


def kernel(H, W_lin, W_val, a, W_out, A0, src, dst):
    W_lin = W_lin.reshape(HEADS, DH, D).transpose(1, 0, 2).reshape(D, D)
    W_val = W_val.reshape(HEADS, DH, D).transpose(1, 0, 2).reshape(D, D)
    W_out = W_out.reshape(D, HEADS, DH).transpose(0, 2, 1).reshape(D, D)
    a = a.T
    e = src.shape[0]
    epad = ((e + 31) // 16) * 16
    srcp = jnp.full((epad,), 4 * N, jnp.int32).at[:e].set(src)
    dstp = jnp.zeros((epad,), jnp.int32).at[:e].set(dst)

    xqT, xqvT = pl.pallas_call(
        _tc_proj_kernel,
        grid=(N // NBLK,),
        in_specs=[pl.BlockSpec((BT, NBLK, D), lambda i: (0, i, 0)),
                  pl.BlockSpec((D, D), lambda i: (0, 0)),
                  pl.BlockSpec((D, D), lambda i: (0, 0))],
        out_specs=[pl.BlockSpec((NBLK, D, BT), lambda i: (i, 0, 0)),
                   pl.BlockSpec((NBLK, 2, D, BT), lambda i: (i, 0, 0, 0))],
        out_shape=[jax.ShapeDtypeStruct((N, D, BT), jnp.float32),
                   jax.ShapeDtypeStruct((N, 2, D, BT), jnp.float32)],
    )(H, W_lin, W_val)

    y_fn = pl.kernel(
        _sc_edge_body,
        out_type=jax.ShapeDtypeStruct((N, ROW), jnp.float32),
        mesh=plsc.VectorSubcoreMesh(core_axis_name="c", subcore_axis_name="s",
                                    num_cores=NC, num_subcores=NS),
        compiler_params=pltpu.CompilerParams(needs_layout_passes=False),
        scratch_types=[
            pltpu.VMEM((epad,), jnp.int32),
            pltpu.VMEM((epad,), jnp.int32),
            pltpu.VMEM((D, 16), jnp.float32),
            pltpu.VMEM((NPW, N + 16), jnp.float32),
            pltpu.VMEM((1, ROW), jnp.float32),
            pltpu.VMEM((2, CH, 2 * ROW), jnp.float32),
            pltpu.VMEM((1, ROW), jnp.float32),
            pltpu.VMEM((1, 256), jnp.float32),
            pltpu.VMEM((2, 16), jnp.int32),
            pltpu.SMEM((16,), jnp.int32),
            pltpu.SemaphoreType.DMA,
        ],
    )
    abct = jnp.broadcast_to(a.reshape(D)[:, None], (D, 16))
    a0p = jnp.concatenate([A0, jnp.zeros((N, 16), jnp.float32)], axis=1)
    y = y_fn(xqT.reshape(N, ROW), xqvT.reshape(N, 2 * ROW), a0p, abct,
             srcp, dstp)

    out = pl.pallas_call(
        _tc_out_kernel,
        grid=(N // NBLK,),
        in_specs=[pl.BlockSpec((NBLK, D, BT), lambda i: (i, 0, 0)),
                  pl.BlockSpec((D, D), lambda i: (0, 0))],
        out_specs=pl.BlockSpec((BT, NBLK, D), lambda i: (0, i, 0)),
        out_shape=jax.ShapeDtypeStruct((BT, N, D), jnp.float32),
    )(y.reshape(N, D, BT), W_out)
    return out


# speedup vs baseline: 2.2081x; 1.1394x over previous
import jax
import jax.numpy as jnp
from jax import lax
from jax.experimental import pallas as pl
from jax.experimental.pallas import tpu as pltpu
from jax.experimental.pallas import tpu_sc as plsc

N = 256
D = 128
HEADS = 4
DH = D // HEADS
BT = 64
ROW = D * BT
NC, NS, L = 2, 16, 16
NW = NC * NS
NPW = N // NW
CH = 2
NBLK = 32


def _tc_proj_kernel(h_ref, wl_ref, wv_ref, xq_ref, xqv_ref):
    for t in range(NBLK):
        hn = h_ref[:, t, :]
        for h in range(HEADS):
            wlh = wl_ref[h * DH:(h + 1) * DH, :]
            wvh = wv_ref[h * DH:(h + 1) * DH, :]
            xq = lax.dot_general(wlh, hn, (((1,), (1,)), ((), ())),
                                 preferred_element_type=jnp.float32)
            xv = lax.dot_general(wvh, hn, (((1,), (1,)), ((), ())),
                                 preferred_element_type=jnp.float32)
            xq_ref[t, :, h * BT:(h + 1) * BT] = xq
            xqv_ref[t, 0, :, h * BT:(h + 1) * BT] = xq
            xqv_ref[t, 1, :, h * BT:(h + 1) * BT] = xv


def _tc_out_kernel(y_ref, wo_ref, out_ref):
    for t in range(NBLK):
        acc = None
        for h in range(HEADS):
            yh = y_ref[t, :, h * BT:(h + 1) * BT]
            woh = wo_ref[:, h * DH:(h + 1) * DH]
            part = lax.dot_general(yh, woh, (((0,), (1,)), ((), ())),
                                   preferred_element_type=jnp.float32)
            acc = part if acc is None else acc + part
        out_ref[:, t, :] = acc


def _sc_edge_body(xq_hbm, xqv_hbm, a0_hbm, ab_hbm, src_hbm, dst_hbm, y_hbm,
                  src_v, dst_v, ab_v, a0r, qrow, kbuf, ybuf, denb, ixst,
                  sm, dsem):
    wid = lax.axis_index("s") * NC + lax.axis_index("c")
    base_node = wid * NPW
    epad = src_v.shape[0]
    lane0 = lax.iota(jnp.int32, 16) == 0
    zi16 = jnp.zeros((16,), jnp.int32)
    zf16v = jnp.zeros((16,), jnp.float32)

    def _lane0i(v):
        return jnp.max(jnp.where(lane0, v, zi16))

    def _lane0f(v):
        return jnp.max(jnp.where(lane0, v, zf16v))

    pltpu.sync_copy(src_hbm, src_v)
    pltpu.sync_copy(dst_hbm, dst_v)
    pltpu.sync_copy(ab_hbm, ab_v)
    pltpu.sync_copy(a0_hbm.at[pl.ds(base_node, NPW)], a0r)

    one16 = jnp.ones((16,), jnp.float32)
    zero16 = jnp.zeros((16,), jnp.float32)

    def cnt_body(i, carry):
        v = src_v[pl.ds(i * 16, 16)]
        c0 = carry[0] + jnp.where(v < base_node, one16, zero16)
        cts = tuple(carry[1 + t] + jnp.where(v == base_node + t, one16, zero16)
                    for t in range(NPW))
        return (c0,) + cts
    zeros_i = tuple(jnp.zeros((16,), jnp.float32) for _ in range(NPW + 1))
    carry = lax.fori_loop(0, epad // 16, cnt_body, zeros_i)
    run = jnp.sum(carry[0]).astype(jnp.int32)
    for t in range(NPW):
        sm[t] = run
        run = run + jnp.sum(carry[1 + t]).astype(jnp.int32)
    sm[NPW] = run

    def node_body(nt, _):
        est = sm[nt]
        deg = sm[nt + 1] - est
        node = base_node + nt
        pltpu.sync_copy(xq_hbm.at[pl.ds(node, 1)], qrow)

        @plsc.parallel_loop(0, ROW // 16, 1, unroll=8)
        def z_body(i):
            ybuf[0, pl.ds(i * 16, 16)] = jnp.zeros((16,), jnp.float32)

        nch = lax.shift_right_logical(deg + (CH - 1), 1)

        def issue(c, slot):
            ixst[slot] = dst_v[pl.ds(est + c * CH, 16)]
            pltpu.async_copy(
                xqv_hbm.at[ixst.at[slot, pl.ds(0, CH)]],
                kbuf.at[slot], dsem)

        def wait(slot):
            pltpu.make_async_copy(
                xqv_hbm.at[ixst.at[slot, pl.ds(0, CH)]],
                kbuf.at[slot], dsem).wait()

        def process(c, slot):
            e0 = c * CH
            s0 = jnp.where(e0 < deg, 1.0, 0.0)
            s1 = jnp.where(e0 + 1 < deg, 1.0, 0.0)
            accs = []
            zf8 = tuple(jnp.zeros((16,), jnp.float32) for _ in range(8))
            for h in range(HEADS):
                def dh_body(d, a8, _h=h):
                    dbase = d * (HEADS * BT) + _h * BT
                    cvec = ab_v[pl.ds((d * HEADS + _h) * 16, 16)]
                    new = []
                    for b in range(BT // 16):
                        off = dbase + 16 * b
                        q = qrow[0, pl.ds(off, 16)]
                        k0 = kbuf[slot, 0, pl.ds(off, 16)]
                        k1 = kbuf[slot, 1, pl.ds(off, 16)]
                        t0 = q + k0
                        t1 = q + k1
                        u0 = jnp.maximum(t0, 0.2 * t0)
                        u1 = jnp.maximum(t1, 0.2 * t1)
                        new.append(a8[2 * b] + cvec * u0)
                        new.append(a8[2 * b + 1] + cvec * u1)
                    return tuple(new)
                a8 = plsc.parallel_loop(0, DH, 1, unroll=2,
                                        carry=zf8)(dh_body)
                accs.append(a8)
            d0 = _lane0i(dst_v[pl.ds(est + e0, 16)])
            d1 = _lane0i(dst_v[pl.ds(est + e0 + 1, 16)])
            w0 = (_lane0f(a0r[nt, pl.ds(d0, 16)]) + 1e-8) * s0
            w1 = (_lane0f(a0r[nt, pl.ds(d1, 16)]) + 1e-8) * s1
            p0 = []
            p1 = []
            for h in range(HEADS):
                for b in range(BT // 16):
                    p0.append(w0 * jnp.exp(jnp.minimum(accs[h][2 * b], 80.0)))
                    p1.append(w1 * jnp.exp(
                        jnp.minimum(accs[h][2 * b + 1], 80.0)))
            for hb in range(16):
                plsc.addupdate(denb.at[0, pl.ds(hb * 16, 16)],
                               p0[hb] + p1[hb])

            @plsc.parallel_loop(0, DH, 1, unroll=2)
            def yd_body(d):
                dbase = d * (HEADS * BT)
                for h in range(HEADS):
                    for b in range(BT // 16):
                        off = dbase + h * BT + 16 * b
                        v0 = kbuf[slot, 0, pl.ds(ROW + off, 16)]
                        v1 = kbuf[slot, 1, pl.ds(ROW + off, 16)]
                        plsc.addupdate(ybuf.at[0, pl.ds(off, 16)],
                                       p0[h * 4 + b] * v0
                                       + p1[h * 4 + b] * v1)

        for hb in range(16):
            denb[0, pl.ds(hb * 16, 16)] = jnp.zeros((16,), jnp.float32)

        @pl.when(nch > 0)
        def _():
            issue(jnp.int32(0), 0)

        def pair_body(pidx, _c):
            c0 = pidx * 2
            wait(0)

            @pl.when(c0 + 1 < nch)
            def _():
                issue(c0 + 1, 1)
            process(c0, 0)

            @pl.when(c0 + 1 < nch)
            def _():
                wait(1)

                @pl.when(c0 + 2 < nch)
                def _():
                    issue(c0 + 2, 0)
                process(c0 + 1, 1)
            return 0
        lax.fori_loop(0, lax.shift_right_logical(nch + 1, 1), pair_body, 0)

        den = [denb[0, pl.ds(hb * 16, 16)] for hb in range(16)]
        rden = [1.0 / jnp.maximum(den[hb], 1e-30) for hb in range(16)]

        @plsc.parallel_loop(0, DH, 1, unroll=4)
        def norm_body(d):
            dbase = d * (HEADS * BT)
            for h in range(HEADS):
                for b in range(BT // 16):
                    off = dbase + h * BT + 16 * b
                    ybuf[0, pl.ds(off, 16)] = (ybuf[0, pl.ds(off, 16)]
                                               * rden[h * 4 + b])
        pltpu.sync_copy(ybuf, y_hbm.at[pl.ds(node, 1)])
        return 0

    lax.fori_loop(0, NPW, node_body, 0)


def kernel(H, W_lin, W_val, a, W_out, A0, src, dst):
    a = a.T
    e = src.shape[0]
    epad = ((e + 31) // 16) * 16
    srcp = jnp.full((epad,), 4 * N, jnp.int32).at[:e].set(src)
    dstp = jnp.zeros((epad,), jnp.int32).at[:e].set(dst)

    xqT, xqvT = pl.pallas_call(
        _tc_proj_kernel,
        grid=(N // NBLK,),
        in_specs=[pl.BlockSpec((BT, NBLK, D), lambda i: (0, i, 0)),
                  pl.BlockSpec((D, D), lambda i: (0, 0)),
                  pl.BlockSpec((D, D), lambda i: (0, 0))],
        out_specs=[pl.BlockSpec((NBLK, DH, HEADS * BT), lambda i: (i, 0, 0)),
                   pl.BlockSpec((NBLK, 2, DH, HEADS * BT),
                                lambda i: (i, 0, 0, 0))],
        out_shape=[jax.ShapeDtypeStruct((N, DH, HEADS * BT), jnp.float32),
                   jax.ShapeDtypeStruct((N, 2, DH, HEADS * BT), jnp.float32)],
    )(H, W_lin, W_val)

    y_fn = pl.kernel(
        _sc_edge_body,
        out_type=jax.ShapeDtypeStruct((N, ROW), jnp.float32),
        mesh=plsc.VectorSubcoreMesh(core_axis_name="c", subcore_axis_name="s",
                                    num_cores=NC, num_subcores=NS),
        compiler_params=pltpu.CompilerParams(needs_layout_passes=False),
        scratch_types=[
            pltpu.VMEM((epad,), jnp.int32),
            pltpu.VMEM((epad,), jnp.int32),
            pltpu.VMEM((D * 16,), jnp.float32),
            pltpu.VMEM((NPW, N + 128), jnp.float32),
            pltpu.VMEM((1, ROW), jnp.float32),
            pltpu.VMEM((2, CH, 2 * ROW), jnp.float32),
            pltpu.VMEM((1, ROW), jnp.float32),
            pltpu.VMEM((1, 256), jnp.float32),
            pltpu.VMEM((2, 16), jnp.int32),
            pltpu.SMEM((16,), jnp.int32),
            pltpu.SemaphoreType.DMA,
        ],
    )
    abct = jnp.broadcast_to(a.reshape(D)[:, None], (D, 16)).reshape(D * 16)
    a0p = jnp.concatenate([A0, jnp.zeros((N, 128), jnp.float32)], axis=1)
    y = y_fn(xqT.reshape(N, ROW), xqvT.reshape(N, 2 * ROW), a0p, abct,
             srcp, dstp)

    out = pl.pallas_call(
        _tc_out_kernel,
        grid=(N // NBLK,),
        in_specs=[pl.BlockSpec((NBLK, DH, HEADS * BT), lambda i: (i, 0, 0)),
                  pl.BlockSpec((D, D), lambda i: (0, 0))],
        out_specs=pl.BlockSpec((BT, NBLK, D), lambda i: (0, i, 0)),
        out_shape=jax.ShapeDtypeStruct((BT, N, D), jnp.float32),
    )(y.reshape(N, DH, HEADS * BT), W_out)
    return out
